# baseline (speedup 1.0000x reference)
import functools

import jax
import jax.numpy as jnp
import numpy as np
from jax import lax
from jax.experimental import pallas as pl
from jax.experimental.pallas import tpu as pltpu
from jax.experimental.pallas import tpu_sc as plsc

VOCAB = 100000
D = 64
HID = 128
B = 1024
CTX = 200

NC = 2
NS = 16
NW = NC * NS

NCH = 2
BCH = B // NCH
B_PER_W = BCH // NW
IDX_PER_W = B_PER_W * CTX
CHUNK = 128
N_CHUNKS = IDX_PER_W // CHUNK

_SEG_NP = (np.arange(IDX_PER_W, dtype=np.int32) // CTX).reshape(N_CHUNKS, CHUNK)


def _sc_embedding_bag(x3, emb, seg):
    mesh = plsc.VectorSubcoreMesh(core_axis_name="c", subcore_axis_name="s")

    @functools.partial(
        pl.kernel,
        mesh=mesh,
        out_type=jax.ShapeDtypeStruct((BCH, D), jnp.float32),
        compiler_params=pltpu.CompilerParams(use_tc_tiling_on_sc=False),
        scratch_types=[
            pltpu.VMEM((N_CHUNKS, CHUNK), jnp.int32),
            pltpu.VMEM((N_CHUNKS, CHUNK), jnp.int32),
            pltpu.VMEM((CHUNK, D), jnp.float32),
            pltpu.VMEM((CHUNK, D), jnp.float32),
            pltpu.VMEM((B_PER_W, D), jnp.float32),
            pltpu.VMEM_SHARED((BCH // NC, D), jnp.float32),
            pltpu.SemaphoreType.DMA,
            pltpu.SemaphoreType.DMA,
        ],
    )
    def k(x_hbm, emb_hbm, seg_hbm, out_hbm, idx_v, seg_v, rows0_v, rows1_v,
          zed_v, acc_sh, sem0, sem1):
        sid = lax.axis_index("s")
        wid = sid * NC + lax.axis_index("c")
        off = sid * B_PER_W
        pltpu.sync_copy(x_hbm.at[wid], idx_v)
        pltpu.sync_copy(seg_hbm, seg_v)

        @pl.loop(0, B_PER_W)
        def _(r):
            @pl.loop(0, D, step=16)
            def _(c0):
                zed_v[r, pl.ds(c0, 16)] = jnp.zeros((16,), jnp.float32)

        @pl.loop(0, N_CHUNKS)
        def _(j):
            @pl.loop(0, CHUNK, step=16)
            def _(c0):
                seg_v[j, pl.ds(c0, 16)] = seg_v[j, pl.ds(c0, 16)] + off

        pltpu.sync_copy(zed_v, acc_sh.at[pl.ds(off, B_PER_W)])

        pltpu.async_copy(emb_hbm.at[idx_v.at[0]], rows0_v, sem0).wait()

        @pl.loop(0, N_CHUNKS - 1, step=2)
        def _(j):
            cp1 = pltpu.async_copy(emb_hbm.at[idx_v.at[j + 1]], rows1_v, sem1)
            pltpu.sync_copy(rows0_v, acc_sh.at[seg_v.at[j]], add=True)
            cp1.wait()
            pltpu.async_copy(emb_hbm.at[idx_v.at[j + 2]], rows0_v, sem0).wait()
            pltpu.sync_copy(rows1_v, acc_sh.at[seg_v.at[j + 1]], add=True)

        pltpu.sync_copy(rows0_v, acc_sh.at[seg_v.at[N_CHUNKS - 1]], add=True)

        pltpu.sync_copy(
            acc_sh.at[pl.ds(off, B_PER_W)],
            out_hbm.at[pl.ds(wid * B_PER_W, B_PER_W)],
        )

    return k(x3, emb, seg)


BN = 4096


def _mlp_body(h_ref, w1_ref, b1_ref, w2_ref, b2_ref, o_ref):
    h1 = (
        jnp.dot(
            h_ref[...] * (1.0 / CTX),
            w1_ref[...],
            preferred_element_type=jnp.float32,
        )
        + b1_ref[...]
    )
    o_ref[...] = (
        jnp.dot(h1, w2_ref[...], preferred_element_type=jnp.float32)
        + b2_ref[...]
    )


_MLP_IN_SPECS = [
    pl.BlockSpec((BCH, D), lambda j: (0, 0)),
    pl.BlockSpec((D, HID), lambda j: (0, 0)),
    pl.BlockSpec((1, HID), lambda j: (0, 0)),
    pl.BlockSpec((HID, BN), lambda j: (0, j)),
    pl.BlockSpec((1, BN), lambda j: (0, j)),
]


def _tc_mlp_band0(h_sum, W1, b1, W2, b2):
    return pl.pallas_call(
        _mlp_body,
        grid=(pl.cdiv(VOCAB, BN),),
        in_specs=_MLP_IN_SPECS,
        out_specs=pl.BlockSpec((BCH, BN), lambda j: (0, j)),
        out_shape=jax.ShapeDtypeStruct((B, VOCAB), jnp.float32),
        compiler_params=pltpu.CompilerParams(
            dimension_semantics=("arbitrary",)
        ),
    )(h_sum, W1, b1.reshape(1, HID), W2, b2.reshape(1, VOCAB))


def _tc_mlp_band1(out, h_sum, W1, b1, W2, b2):
    def body(big_ref, h_ref, w1_ref, b1_ref, w2_ref, b2_ref, o_ref):
        del big_ref
        _mlp_body(h_ref, w1_ref, b1_ref, w2_ref, b2_ref, o_ref)

    return pl.pallas_call(
        body,
        grid=(pl.cdiv(VOCAB, BN),),
        in_specs=[pl.BlockSpec(memory_space=pl.ANY)] + _MLP_IN_SPECS,
        out_specs=pl.BlockSpec((BCH, BN), lambda j: (1, j)),
        out_shape=jax.ShapeDtypeStruct((B, VOCAB), jnp.float32),
        input_output_aliases={0: 0},
        compiler_params=pltpu.CompilerParams(
            dimension_semantics=("arbitrary",)
        ),
    )(out, h_sum, W1, b1.reshape(1, HID), W2, b2.reshape(1, VOCAB))


def kernel(x, emb, W1, b1, W2, b2):
    xr = x.reshape(NCH, NW, N_CHUNKS, CHUNK)
    seg = jnp.asarray(_SEG_NP)
    h0 = _sc_embedding_bag(xr[0], emb, seg)
    h1 = _sc_embedding_bag(xr[1], emb, seg)
    out = _tc_mlp_band0(h0, W1, b1, W2, b2)
    out = _tc_mlp_band1(out, h1, W1, b1, W2, b2)
    return out


# speedup vs baseline: 1.0464x; 1.0464x over previous
import functools

import jax
import jax.numpy as jnp
import numpy as np
from jax import lax
from jax.experimental import pallas as pl
from jax.experimental.pallas import tpu as pltpu
from jax.experimental.pallas import tpu_sc as plsc

VOCAB = 100000
D = 64
HID = 128
B = 1024
CTX = 200

NC = 2
NS = 16
NW = NC * NS

B_PER_W = B // NW
IDX_PER_W = B_PER_W * CTX
CHUNK = 128
N_CHUNKS = IDX_PER_W // CHUNK

_SEG_NP = (np.arange(IDX_PER_W, dtype=np.int32) // CTX).reshape(N_CHUNKS, CHUNK)


def _sc_embedding_bag(x3, emb, seg):
    mesh = plsc.VectorSubcoreMesh(core_axis_name="c", subcore_axis_name="s")

    @functools.partial(
        pl.kernel,
        mesh=mesh,
        out_type=jax.ShapeDtypeStruct((B, D), jnp.float32),
        compiler_params=pltpu.CompilerParams(use_tc_tiling_on_sc=False),
        scratch_types=[
            pltpu.VMEM((N_CHUNKS, CHUNK), jnp.int32),
            pltpu.VMEM((N_CHUNKS, CHUNK), jnp.int32),
            pltpu.VMEM((CHUNK, D), jnp.float32),
            pltpu.VMEM((CHUNK, D), jnp.float32),
            pltpu.VMEM((B_PER_W, D), jnp.float32),
            pltpu.VMEM_SHARED((B // NC, D), jnp.float32),
            pltpu.SemaphoreType.DMA,
            pltpu.SemaphoreType.DMA,
        ],
    )
    def k(x_hbm, emb_hbm, seg_hbm, out_hbm, idx_v, seg_v, rows0_v, rows1_v,
          zed_v, acc_sh, sem0, sem1):
        sid = lax.axis_index("s")
        wid = sid * NC + lax.axis_index("c")
        off = sid * B_PER_W
        pltpu.sync_copy(x_hbm.at[wid], idx_v)
        pltpu.sync_copy(seg_hbm, seg_v)

        @pl.loop(0, B_PER_W)
        def _(r):
            @pl.loop(0, D, step=16)
            def _(c0):
                zed_v[r, pl.ds(c0, 16)] = jnp.zeros((16,), jnp.float32)

        @pl.loop(0, N_CHUNKS)
        def _(j):
            @pl.loop(0, CHUNK, step=16)
            def _(c0):
                seg_v[j, pl.ds(c0, 16)] = seg_v[j, pl.ds(c0, 16)] + off

        pltpu.sync_copy(zed_v, acc_sh.at[pl.ds(off, B_PER_W)])

        pltpu.async_copy(emb_hbm.at[idx_v.at[0]], rows0_v, sem0).wait()

        @pl.loop(0, N_CHUNKS - 2, step=2)
        def _(j):
            cp1 = pltpu.async_copy(emb_hbm.at[idx_v.at[j + 1]], rows1_v, sem1)
            pltpu.sync_copy(rows0_v, acc_sh.at[seg_v.at[j]], add=True)
            cp1.wait()
            pltpu.async_copy(emb_hbm.at[idx_v.at[j + 2]], rows0_v, sem0).wait()
            pltpu.sync_copy(rows1_v, acc_sh.at[seg_v.at[j + 1]], add=True)

        cp1 = pltpu.async_copy(
            emb_hbm.at[idx_v.at[N_CHUNKS - 1]], rows1_v, sem1)
        pltpu.sync_copy(rows0_v, acc_sh.at[seg_v.at[N_CHUNKS - 2]], add=True)
        cp1.wait()
        pltpu.sync_copy(rows1_v, acc_sh.at[seg_v.at[N_CHUNKS - 1]], add=True)

        pltpu.sync_copy(
            acc_sh.at[pl.ds(off, B_PER_W)],
            out_hbm.at[pl.ds(wid * B_PER_W, B_PER_W)],
        )

    return k(x3, emb, seg)


BN = 4096


def _mlp_body(h_ref, w1_ref, b1_ref, w2_ref, b2_ref, o_ref):
    h1 = (
        jnp.dot(
            h_ref[...] * (1.0 / CTX),
            w1_ref[...],
            preferred_element_type=jnp.float32,
        )
        + b1_ref[...]
    )
    o_ref[...] = (
        jnp.dot(h1, w2_ref[...], preferred_element_type=jnp.float32)
        + b2_ref[...]
    )


def _tc_mlp(h_sum, W1, b1, W2, b2):
    return pl.pallas_call(
        _mlp_body,
        grid=(pl.cdiv(VOCAB, BN),),
        in_specs=[
            pl.BlockSpec((B, D), lambda j: (0, 0)),
            pl.BlockSpec((D, HID), lambda j: (0, 0)),
            pl.BlockSpec((1, HID), lambda j: (0, 0)),
            pl.BlockSpec((HID, BN), lambda j: (0, j)),
            pl.BlockSpec((1, BN), lambda j: (0, j)),
        ],
        out_specs=pl.BlockSpec((B, BN), lambda j: (0, j)),
        out_shape=jax.ShapeDtypeStruct((B, VOCAB), jnp.float32),
        compiler_params=pltpu.CompilerParams(
            dimension_semantics=("arbitrary",)
        ),
    )(h_sum, W1, b1.reshape(1, HID), W2, b2.reshape(1, VOCAB))


def kernel(x, emb, W1, b1, W2, b2):
    xr = x.reshape(NW, N_CHUNKS, CHUNK)
    seg = jnp.asarray(_SEG_NP)
    h = _sc_embedding_bag(xr, emb, seg)
    return _tc_mlp(h, W1, b1, W2, b2)


# speedup vs baseline: 1.0500x; 1.0035x over previous
import functools

import jax
import jax.numpy as jnp
import numpy as np
from jax import lax
from jax.experimental import pallas as pl
from jax.experimental.pallas import tpu as pltpu
from jax.experimental.pallas import tpu_sc as plsc

VOCAB = 100000
D = 64
HID = 128
B = 1024
CTX = 200

NC = 2
NS = 16
NW = NC * NS

B_PER_W = B // NW
IDX_PER_W = B_PER_W * CTX
CHUNK = 128
N_CHUNKS = IDX_PER_W // CHUNK

_SEG_NP = (np.arange(IDX_PER_W, dtype=np.int32) // CTX).reshape(N_CHUNKS, CHUNK)


def _sc_embedding_bag(x3, emb, seg):
    mesh = plsc.VectorSubcoreMesh(core_axis_name="c", subcore_axis_name="s")

    @functools.partial(
        pl.kernel,
        mesh=mesh,
        out_type=jax.ShapeDtypeStruct((B, D), jnp.float32),
        compiler_params=pltpu.CompilerParams(use_tc_tiling_on_sc=False),
        scratch_types=[
            pltpu.VMEM((N_CHUNKS, CHUNK), jnp.int32),
            pltpu.VMEM((N_CHUNKS, CHUNK), jnp.int32),
            pltpu.VMEM((CHUNK, D), jnp.float32),
            pltpu.VMEM((CHUNK, D), jnp.float32),
            pltpu.VMEM((B_PER_W, D), jnp.float32),
            pltpu.VMEM_SHARED((B // NC, D), jnp.float32),
            pltpu.SemaphoreType.DMA,
            pltpu.SemaphoreType.DMA,
        ],
    )
    def k(x_hbm, emb_hbm, seg_hbm, out_hbm, idx_v, seg_v, rows0_v, rows1_v,
          zed_v, acc_sh, sem0, sem1):
        sid = lax.axis_index("s")
        wid = sid * NC + lax.axis_index("c")
        off = sid * B_PER_W
        pltpu.sync_copy(x_hbm.at[wid], idx_v)
        pltpu.sync_copy(seg_hbm, seg_v)

        @pl.loop(0, B_PER_W)
        def _(r):
            @pl.loop(0, D, step=16)
            def _(c0):
                zed_v[r, pl.ds(c0, 16)] = jnp.zeros((16,), jnp.float32)

        @pl.loop(0, N_CHUNKS)
        def _(j):
            @pl.loop(0, CHUNK, step=16)
            def _(c0):
                seg_v[j, pl.ds(c0, 16)] = seg_v[j, pl.ds(c0, 16)] + off

        pltpu.sync_copy(zed_v, acc_sh.at[pl.ds(off, B_PER_W)])

        pltpu.async_copy(emb_hbm.at[idx_v.at[0]], rows0_v, sem0).wait()

        @pl.loop(0, N_CHUNKS - 2, step=2)
        def _(j):
            cp1 = pltpu.async_copy(emb_hbm.at[idx_v.at[j + 1]], rows1_v, sem1)
            pltpu.sync_copy(rows0_v, acc_sh.at[seg_v.at[j]], add=True)
            cp1.wait()
            pltpu.async_copy(emb_hbm.at[idx_v.at[j + 2]], rows0_v, sem0).wait()
            pltpu.sync_copy(rows1_v, acc_sh.at[seg_v.at[j + 1]], add=True)

        cp1 = pltpu.async_copy(
            emb_hbm.at[idx_v.at[N_CHUNKS - 1]], rows1_v, sem1)
        pltpu.sync_copy(rows0_v, acc_sh.at[seg_v.at[N_CHUNKS - 2]], add=True)
        cp1.wait()
        pltpu.sync_copy(rows1_v, acc_sh.at[seg_v.at[N_CHUNKS - 1]], add=True)

        pltpu.sync_copy(
            acc_sh.at[pl.ds(off, B_PER_W)],
            out_hbm.at[pl.ds(wid * B_PER_W, B_PER_W)],
        )

    return k(x3, emb, seg)


BN = 4096


def _mlp_body(h_ref, w1_ref, b1_ref, w2_ref, b2_ref, o_ref):
    h1 = (
        jnp.dot(
            h_ref[...] * (1.0 / CTX),
            w1_ref[...],
            preferred_element_type=jnp.float32,
        )
        + b1_ref[...]
    )
    o_ref[...] = (
        jnp.dot(
            h1.astype(jnp.bfloat16),
            w2_ref[...],
            preferred_element_type=jnp.float32,
        )
        + b2_ref[...]
    )


def _tc_mlp(h_sum, W1, b1, W2, b2):
    return pl.pallas_call(
        _mlp_body,
        grid=(pl.cdiv(VOCAB, BN),),
        in_specs=[
            pl.BlockSpec((B, D), lambda j: (0, 0)),
            pl.BlockSpec((D, HID), lambda j: (0, 0)),
            pl.BlockSpec((1, HID), lambda j: (0, 0)),
            pl.BlockSpec((HID, BN), lambda j: (0, j)),
            pl.BlockSpec((1, BN), lambda j: (0, j)),
        ],
        out_specs=pl.BlockSpec((B, BN), lambda j: (0, j)),
        out_shape=jax.ShapeDtypeStruct((B, VOCAB), jnp.float32),
        compiler_params=pltpu.CompilerParams(
            dimension_semantics=("arbitrary",)
        ),
    )(h_sum, W1, b1.reshape(1, HID),
      W2.astype(jnp.bfloat16), b2.reshape(1, VOCAB))


def kernel(x, emb, W1, b1, W2, b2):
    xr = x.reshape(NW, N_CHUNKS, CHUNK)
    seg = jnp.asarray(_SEG_NP)
    h = _sc_embedding_bag(xr, emb, seg)
    return _tc_mlp(h, W1, b1, W2, b2)
